# baseline (speedup 1.0000x reference)
import functools

import jax
import jax.numpy as jnp
from jax import lax
from jax.experimental import pallas as pl
from jax.experimental.pallas import tpu as pltpu
from jax.experimental.pallas import tpu_sc as plsc

D_MODEL = 768


@jax.jit
def _embed(idx, W_E):
    (N,) = idx.shape
    info = plsc.get_sparse_core_info()
    NW = info.num_cores * info.num_subcores
    n_per_w = N // NW
    C = 64
    NBUF = 2
    n_chunks = n_per_w // C

    mesh = plsc.VectorSubcoreMesh(core_axis_name="c", subcore_axis_name="s")

    @functools.partial(
        pl.kernel,
        mesh=mesh,
        out_type=jax.ShapeDtypeStruct((N, D_MODEL), jnp.float32),
        scratch_types=[pltpu.VMEM((C,), jnp.int32) for _ in range(n_chunks)]
        + [
            pltpu.VMEM((NBUF, C, D_MODEL), jnp.float32),
            pltpu.SemaphoreType.DMA,
        ]
        + [pltpu.SemaphoreType.DMA] * (2 * NBUF),
    )
    def embed_k(tok_hbm, w_hbm, out_hbm, *refs):
        idx_c = refs[:n_chunks]
        rows_v = refs[n_chunks]
        isem = refs[n_chunks + 1]
        gsem = refs[n_chunks + 2 : n_chunks + 2 + NBUF]
        ssem = refs[n_chunks + 2 + NBUF :]
        wid = lax.axis_index("s") * info.num_cores + lax.axis_index("c")
        base = wid * n_per_w
        icopies = [
            pltpu.async_copy(tok_hbm.at[pl.ds(base + i * C, C)], idx_c[i], isem)
            for i in range(n_chunks)
        ]
        for c in icopies:
            c.wait()

        def start_gather(i, b):
            return pltpu.async_copy(
                w_hbm.at[idx_c[i]], rows_v.at[b], gsem[b]
            )

        def start_scatter(i, b):
            return pltpu.async_copy(
                rows_v.at[b], out_hbm.at[pl.ds(base + i * C, C)], ssem[b]
            )

        g = [None] * NBUF
        s = [None] * NBUF
        for i in range(n_chunks):
            b = i % NBUF
            if s[b] is not None:
                s[b].wait()
            g[b] = start_gather(i, b)
            if i >= NBUF - 1:
                j = i - (NBUF - 1)
                pb = j % NBUF
                g[pb].wait()
                s[pb] = start_scatter(j, pb)
        for j in range(n_chunks - (NBUF - 1), n_chunks):
            pb = j % NBUF
            g[pb].wait()
            s[pb] = start_scatter(j, pb)
        for b in range(NBUF):
            if s[b] is not None:
                s[b].wait()

    return embed_k(idx, W_E)


def kernel(tokens, W_E):
    B, S = tokens.shape
    idx = tokens.reshape(B * S).astype(jnp.int32)
    out = _embed(idx, W_E)
    return out.reshape(B, S, D_MODEL)


# speedup vs baseline: 1.0189x; 1.0189x over previous
import functools

import jax
import jax.numpy as jnp
from jax import lax
from jax.experimental import pallas as pl
from jax.experimental.pallas import tpu as pltpu
from jax.experimental.pallas import tpu_sc as plsc

D_MODEL = 768


@jax.jit
def _embed(idx, W_E):
    (N,) = idx.shape
    info = plsc.get_sparse_core_info()
    NW = info.num_cores * info.num_subcores
    n_per_w = N // NW
    C = 128
    n_chunks = n_per_w // C

    mesh = plsc.VectorSubcoreMesh(core_axis_name="c", subcore_axis_name="s")

    @functools.partial(
        pl.kernel,
        mesh=mesh,
        out_type=jax.ShapeDtypeStruct((N, D_MODEL), jnp.float32),
        scratch_types=[
            pltpu.VMEM((n_per_w,), jnp.int32),
            pltpu.VMEM((C, D_MODEL), jnp.float32),
            pltpu.SemaphoreType.DMA,
        ],
    )
    def embed_k(tok_hbm, w_hbm, out_hbm, idx_v, rows_v, gsem):
        wid = lax.axis_index("s") * info.num_cores + lax.axis_index("c")
        base = wid * n_per_w
        pltpu.sync_copy(tok_hbm.at[pl.ds(base, n_per_w)], idx_v)

        def body(i, _):
            pltpu.async_copy(
                w_hbm.at[idx_v.at[pl.ds(i * C, C)]], rows_v, gsem
            ).wait()
            pltpu.sync_copy(rows_v, out_hbm.at[pl.ds(base + i * C, C)])
            return 0

        lax.fori_loop(0, n_chunks, body, 0)

    return embed_k(idx, W_E)


def kernel(tokens, W_E):
    B, S = tokens.shape
    idx = tokens.reshape(B * S).astype(jnp.int32)
    out = _embed(idx, W_E)
    return out.reshape(B, S, D_MODEL)


# speedup vs baseline: 1.0202x; 1.0013x over previous
import functools

import jax
import jax.numpy as jnp
from jax import lax
from jax.experimental import pallas as pl
from jax.experimental.pallas import tpu as pltpu
from jax.experimental.pallas import tpu_sc as plsc

D_MODEL = 768


@jax.jit
def _embed(tokens, W_E):
    B, S = tokens.shape
    N = B * S
    info = plsc.get_sparse_core_info()
    NW = info.num_cores * info.num_subcores
    n_per_w = N // NW
    w_per_row = S // n_per_w
    C = 128
    n_chunks = n_per_w // C

    mesh = plsc.VectorSubcoreMesh(core_axis_name="c", subcore_axis_name="s")

    @functools.partial(
        pl.kernel,
        mesh=mesh,
        out_type=jax.ShapeDtypeStruct((N, D_MODEL), jnp.float32),
        scratch_types=[
            pltpu.VMEM((n_per_w,), jnp.int32),
            pltpu.VMEM((C, D_MODEL), jnp.float32),
            pltpu.SemaphoreType.DMA,
        ],
    )
    def embed_k(tok_hbm, w_hbm, out_hbm, idx_v, rows_v, gsem):
        wid = lax.axis_index("s") * info.num_cores + lax.axis_index("c")
        base = wid * n_per_w
        r = wid // w_per_row
        col = (wid % w_per_row) * n_per_w
        pltpu.sync_copy(tok_hbm.at[r, pl.ds(col, n_per_w)], idx_v)

        def body(i, _):
            pltpu.async_copy(
                w_hbm.at[idx_v.at[pl.ds(i * C, C)]], rows_v, gsem
            ).wait()
            pltpu.sync_copy(rows_v, out_hbm.at[pl.ds(base + i * C, C)])
            return 0

        lax.fori_loop(0, n_chunks, body, 0)

    return embed_k(tokens, W_E)


def kernel(tokens, W_E):
    B, S = tokens.shape
    out = _embed(tokens.astype(jnp.int32), W_E)
    return out.reshape(B, S, D_MODEL)


# speedup vs baseline: 1.0229x; 1.0027x over previous
import functools

import jax
import jax.numpy as jnp
from jax import lax
from jax.experimental import pallas as pl
from jax.experimental.pallas import tpu as pltpu
from jax.experimental.pallas import tpu_sc as plsc

D_MODEL = 768


@jax.jit
def _embed(tokens, W_E):
    B, S = tokens.shape
    N = B * S
    info = plsc.get_sparse_core_info()
    NW = info.num_cores * info.num_subcores
    n_per_w = N // NW
    w_per_row = S // n_per_w
    C = 32
    NBUF = 4
    n_chunks = n_per_w // C

    mesh = plsc.VectorSubcoreMesh(core_axis_name="c", subcore_axis_name="s")

    @functools.partial(
        pl.kernel,
        mesh=mesh,
        out_type=jax.ShapeDtypeStruct((N, D_MODEL), jnp.float32),
        scratch_types=[
            pltpu.VMEM((n_per_w,), jnp.int32),
            pltpu.VMEM((NBUF, C, D_MODEL), jnp.float32),
        ]
        + [pltpu.SemaphoreType.DMA] * (2 * NBUF),
    )
    def embed_k(tok_hbm, w_hbm, out_hbm, idx_v, rows_v, *sems):
        gsem = sems[:NBUF]
        ssem = sems[NBUF:]
        wid = lax.axis_index("s") * info.num_cores + lax.axis_index("c")
        base = wid * n_per_w
        r = wid // w_per_row
        col = (wid % w_per_row) * n_per_w
        pltpu.sync_copy(tok_hbm.at[r, pl.ds(col, n_per_w)], idx_v)

        def start_gather(i, b):
            return pltpu.async_copy(
                w_hbm.at[idx_v.at[pl.ds(i * C, C)]], rows_v.at[b], gsem[b]
            )

        def start_scatter(i, b):
            return pltpu.async_copy(
                rows_v.at[b], out_hbm.at[pl.ds(base + i * C, C)], ssem[b]
            )

        g = [None] * NBUF
        s = [None] * NBUF
        for i in range(n_chunks):
            b = i % NBUF
            if s[b] is not None:
                s[b].wait()
            g[b] = start_gather(i, b)
            if i >= NBUF - 1:
                j = i - (NBUF - 1)
                pb = j % NBUF
                g[pb].wait()
                s[pb] = start_scatter(j, pb)
        for j in range(n_chunks - (NBUF - 1), n_chunks):
            pb = j % NBUF
            g[pb].wait()
            s[pb] = start_scatter(j, pb)
        for b in range(NBUF):
            if s[b] is not None:
                s[b].wait()

    return embed_k(tokens, W_E)


def kernel(tokens, W_E):
    B, S = tokens.shape
    out = _embed(tokens.astype(jnp.int32), W_E)
    return out.reshape(B, S, D_MODEL)
